# baseline (speedup 1.0000x reference)
import functools

import jax
import jax.numpy as jnp
from jax import lax
from jax.experimental import pallas as pl
from jax.experimental.pallas import tpu as pltpu
from jax.experimental.pallas import tpu_sc as plsc

M = 18432
N_CODES = 8192
DIM = 256
BETA = 0.25

BM = 1024
BN = 2048
N_CHUNKS = N_CODES // BN
GPC = BN // 128


def _argmin_body(xn_ref, en_ref, jf_ref, xm2_ref, ebf_ref, idx_ref, dsum_ref):
    xm2 = xm2_ref[...]
    xn = xn_ref[...]

    xnb = jnp.broadcast_to(xn, (BM, 128))
    best = jnp.full((BM, 128), jnp.inf, dtype=jnp.float32)
    bjf = jnp.zeros((BM, 128), dtype=jnp.float32)
    for c in range(N_CHUNKS):
        ebf = ebf_ref[pl.ds(c * BN, BN), :]
        nmm2 = lax.dot_general(
            xm2, ebf, (((1,), (1,)), ((), ())),
            preferred_element_type=jnp.float32)
        for g in range(GPC):
            en_g = en_ref[pl.ds(c * BN + g * 128, 128)]
            dg = (xnb + en_g[None, :]) + nmm2[:, g * 128:(g + 1) * 128]
            m = dg < best
            best = jnp.where(m, dg, best)
            bjf = jnp.where(m, float(c * BN + g * 128), bjf)

    lanef = lax.broadcasted_iota(jnp.int32, (BM, 128), 1).astype(jnp.float32)
    fbest = jnp.min(best, axis=1)
    bidx = jnp.min(jnp.where(best == fbest[:, None], bjf + lanef, jnp.inf),
                   axis=1).astype(jnp.int32)

    idx_ref[...] = bidx

    @pl.when(pl.program_id(0) == 0)
    def _():
        dsum_ref[...] = jnp.zeros((1, 1), jnp.float32)

    dsum_ref[...] += jnp.sum(fbest).reshape(1, 1)


def _tc_argmin(xn, en, jf, xbf, ebf):
    grid = (M // BM,)
    return pl.pallas_call(
        _argmin_body,
        grid=grid,
        in_specs=[
            pl.BlockSpec((BM, 1), lambda i: (i, 0)),
            pl.BlockSpec((N_CODES,), lambda i: (0,)),
            pl.BlockSpec((N_CODES,), lambda i: (0,)),
            pl.BlockSpec((BM, DIM), lambda i: (i, 0)),
            pl.BlockSpec((N_CODES, DIM), lambda i: (0, 0)),
        ],
        out_specs=[
            pl.BlockSpec((BM,), lambda i: (i,)),
            pl.BlockSpec((1, 1), lambda i: (0, 0)),
        ],
        out_shape=[
            jax.ShapeDtypeStruct((M,), jnp.int32),
            jax.ShapeDtypeStruct((1, 1), jnp.float32),
        ],
    )(xn, en, jf, xbf, ebf)


_NW = 32
_BPW = M // _NW
_CH = 144
_NCH = _BPW // _CH


@functools.cache
def _sc_gather_kernel():
    @functools.partial(
        pl.kernel,
        out_type=jax.ShapeDtypeStruct((M, DIM), jnp.float32),
        mesh=plsc.VectorSubcoreMesh(core_axis_name="c", subcore_axis_name="s"),
        scratch_types=[
            pltpu.VMEM((_BPW,), jnp.int32),
            pltpu.VMEM((2, _CH, DIM), jnp.float32),
            pltpu.SemaphoreType.DMA,
            pltpu.SemaphoreType.DMA,
            pltpu.SemaphoreType.DMA,
        ],
    )
    def _sc_gather(emb_hbm, idx_hbm, out_hbm, idx_v, rows_v, g_sem, s_sem0,
                   s_sem1):
        wid = lax.axis_index("s") * 2 + lax.axis_index("c")
        base = wid * _BPW
        s_sems = (s_sem0, s_sem1)
        pltpu.sync_copy(idx_hbm.at[pl.ds(base, _BPW)], idx_v)
        gathers = [None] * _NCH
        scatters = [None] * _NCH
        gathers[0] = pltpu.async_copy(
            emb_hbm.at[idx_v.at[pl.ds(0, _CH)]], rows_v.at[0], g_sem)
        for c in range(_NCH):
            buf = c % 2
            gathers[c].wait()
            if c + 1 < _NCH:
                if c - 1 >= 0:
                    scatters[c - 1].wait()
                gathers[c + 1] = pltpu.async_copy(
                    emb_hbm.at[idx_v.at[pl.ds((c + 1) * _CH, _CH)]],
                    rows_v.at[1 - buf], g_sem)
            scatters[c] = pltpu.async_copy(
                rows_v.at[buf], out_hbm.at[pl.ds(base + c * _CH, _CH)],
                s_sems[buf])
        scatters[_NCH - 2].wait()
        scatters[_NCH - 1].wait()

    return _sc_gather


def kernel(x, embeddings):
    xf = x.reshape(-1, DIM)
    xn = jnp.sum(xf ** 2, axis=1, keepdims=True)
    en = jnp.sum(embeddings ** 2, axis=1)
    xm2 = (xf * -2.0).astype(jnp.bfloat16)
    ebf = embeddings.astype(jnp.bfloat16)
    jf = jnp.arange(N_CODES, dtype=jnp.float32)
    idx, dsum = _tc_argmin(xn, en, jf, xm2, ebf)
    quantized = _sc_gather_kernel()(embeddings, idx)
    loss = (BETA / (M * DIM)) * dsum[0, 0]
    return quantized.reshape(x.shape), loss
